# Initial kernel scaffold; baseline (speedup 1.0000x reference)
import functools

import jax
import jax.numpy as jnp
from jax import lax
from jax.experimental import pallas as pl
from jax.experimental.pallas import tpu as pltpu
from jax.experimental.pallas import tpu_sc as plsc

TOP_K = 2
NUM_EXPERTS = 16
T = 4096
D = 2048
N = T * TOP_K

NC, NS, L = 2, 16, 16
NW = NC * NS
CHUNK = N // NW
CVECS = CHUNK // L
TOK_W = T // NW
GT = 8
NCHUNKS = TOK_W // GT


def _wid():
    return lax.axis_index("s") * NC + lax.axis_index("c")


_mesh = plsc.VectorSubcoreMesh(core_axis_name="c", subcore_axis_name="s")


@functools.partial(
    pl.kernel,
    out_type=(
        jax.ShapeDtypeStruct((N,), jnp.int32),
        jax.ShapeDtypeStruct((NW, L), jnp.int32),
    ),
    mesh=_mesh,
    scratch_types=[
        pltpu.VMEM((CHUNK,), jnp.int32),
        pltpu.VMEM((CHUNK,), jnp.int32),
        pltpu.VMEM((L,), jnp.int32),
    ],
)
def _rank_kernel(meta_hbm, ranks_hbm, hist_hbm, ev_v, rank_v, hist_v):
    w = _wid()
    pltpu.sync_copy(meta_hbm.at[pl.ds(w * CHUNK, CHUNK)], ev_v)

    def body(v, c):
        ev = ev_v[pl.ds(v * L, L)]
        r = jnp.zeros((L,), jnp.int32)
        newc = []
        for e in range(NUM_EXPERTS):
            m = ev == e
            mi = m.astype(jnp.int32)
            cs = plsc.cumsum(mi)
            r = jnp.where(m, cs - 1 + c[e], r)
            newc.append(c[e] + jnp.sum(mi))
        rank_v[pl.ds(v * L, L)] = r
        return tuple(newc)

    c0 = tuple(jnp.int32(0) for _ in range(NUM_EXPERTS))
    cend = lax.fori_loop(0, CVECS, body, c0)
    hist_v[...] = jnp.stack(cend)
    pltpu.sync_copy(rank_v, ranks_hbm.at[pl.ds(w * CHUNK, CHUNK)])
    pltpu.sync_copy(hist_v, hist_hbm.at[w])


@functools.partial(
    pl.kernel,
    out_type=jax.ShapeDtypeStruct((T, D), jnp.float32),
    mesh=_mesh,
    scratch_types=[
        pltpu.VMEM((NW * L,), jnp.int32),
        pltpu.VMEM((L,), jnp.int32),
        pltpu.VMEM((CHUNK,), jnp.int32),
        pltpu.VMEM((CHUNK,), jnp.int32),
        pltpu.VMEM((CHUNK,), jnp.int32),
        pltpu.VMEM((2 * GT, D), jnp.float32),
        pltpu.VMEM((GT, D), jnp.float32),
        pltpu.SemaphoreType.DMA,
    ],
)
def _combine_kernel(input_hbm, meta_hbm, ranks_hbm, hist_hbm, out_hbm,
                    hist_v, base_v, ev_v, rank_v, idx_v, in_v, out_v, sem):
    w = _wid()
    pltpu.sync_copy(hist_hbm, hist_v)
    pltpu.sync_copy(meta_hbm.at[pl.ds(w * CHUNK, CHUNK)], ev_v)
    pltpu.sync_copy(ranks_hbm.at[pl.ds(w * CHUNK, CHUNK)], rank_v)

    def hbody(wp, carry):
        pref, tot = carry
        hv = hist_v[pl.ds(wp * L, L)]
        pref = pref + jnp.where(wp < w, hv, 0)
        return pref, tot + hv

    zero = jnp.zeros((L,), jnp.int32)
    pref, tot = lax.fori_loop(0, NW, hbody, (zero, zero))
    offset = plsc.cumsum(tot) - tot
    base_v[...] = offset + pref

    def ibody(v, _):
        ev = ev_v[pl.ds(v * L, L)]
        bg = plsc.load_gather(base_v, [ev])
        idx_v[pl.ds(v * L, L)] = bg + rank_v[pl.ds(v * L, L)]
        return 0

    lax.fori_loop(0, CVECS, ibody, 0)

    def gbody(g, _):
        pltpu.async_copy(
            input_hbm.at[idx_v.at[pl.ds(g * 2 * GT, 2 * GT)]], in_v, sem
        ).wait()

        def sbody(s, _):
            i = s >> 7
            vv = (s & 127) * L
            a = in_v[2 * i, pl.ds(vv, L)]
            b = in_v[2 * i + 1, pl.ds(vv, L)]
            out_v[i, pl.ds(vv, L)] = a + b
            return 0

        lax.fori_loop(0, GT * (D // L), sbody, 0)
        pltpu.sync_copy(out_v, out_hbm.at[pl.ds(w * TOK_W + g * GT, GT)])
        return 0

    lax.fori_loop(0, NCHUNKS, gbody, 0)


def kernel(input_tensor, expert_metadata, expert_mapping, expert_locals):
    del expert_mapping, expert_locals
    meta = expert_metadata.reshape(-1).astype(jnp.int32)
    ranks, hist = _rank_kernel(meta)
    return _combine_kernel(input_tensor, meta, ranks, hist.reshape(-1))


# speedup vs baseline: 2.2850x; 2.2850x over previous
import functools

import jax
import jax.numpy as jnp
from jax import lax
from jax.experimental import pallas as pl
from jax.experimental.pallas import tpu as pltpu
from jax.experimental.pallas import tpu_sc as plsc

TOP_K = 2
NUM_EXPERTS = 16
T = 4096
D = 2048
N = T * TOP_K

NC, NS, L = 2, 16, 16
NW = NC * NS
CHUNK = N // NW
CVECS = CHUNK // L
TOK_W = T // NW
GT = 8
NCHUNKS = TOK_W // GT


def _wid():
    return lax.axis_index("s") * NC + lax.axis_index("c")


_mesh = plsc.VectorSubcoreMesh(core_axis_name="c", subcore_axis_name="s")


@functools.partial(
    pl.kernel,
    out_type=(
        jax.ShapeDtypeStruct((N,), jnp.int32),
        jax.ShapeDtypeStruct((NW, L), jnp.int32),
    ),
    mesh=_mesh,
    compiler_params=pltpu.CompilerParams(needs_layout_passes=False),
    scratch_types=[
        pltpu.VMEM((CHUNK,), jnp.int32),
        pltpu.VMEM((CHUNK,), jnp.int32),
        pltpu.VMEM((L,), jnp.int32),
    ],
)
def _rank_kernel(meta_hbm, ranks_hbm, hist_hbm, ev_v, rank_v, hist_v):
    w = _wid()
    pltpu.sync_copy(meta_hbm.at[pl.ds(w * CHUNK, CHUNK)], ev_v)
    hist_v[...] = jnp.zeros((L,), jnp.int32)

    def body(v, _):
        ev = ev_v[pl.ds(v * L, L)]
        carry = plsc.load_gather(hist_v, [ev])
        r = jnp.zeros((L,), jnp.int32)
        for e in range(NUM_EXPERTS):
            m = ev == e
            cs = plsc.cumsum(m.astype(jnp.int32))
            r = jnp.where(m, cs - 1, r)
        rank_v[pl.ds(v * L, L)] = carry + r
        plsc.addupdate_scatter(hist_v, [ev], jnp.ones((L,), jnp.int32))
        return 0

    lax.fori_loop(0, CVECS, body, 0)
    pltpu.sync_copy(rank_v, ranks_hbm.at[pl.ds(w * CHUNK, CHUNK)])
    pltpu.sync_copy(hist_v, hist_hbm.at[w])


@functools.partial(
    pl.kernel,
    out_type=jax.ShapeDtypeStruct((T, D), jnp.float32),
    mesh=_mesh,
    compiler_params=pltpu.CompilerParams(needs_layout_passes=False),
    scratch_types=[
        pltpu.VMEM((NW * L,), jnp.int32),
        pltpu.VMEM((L,), jnp.int32),
        pltpu.VMEM((CHUNK,), jnp.int32),
        pltpu.VMEM((CHUNK,), jnp.int32),
        pltpu.VMEM((CHUNK,), jnp.int32),
        pltpu.VMEM((2 * GT, D), jnp.float32),
        pltpu.VMEM((GT, D), jnp.float32),
        pltpu.SemaphoreType.DMA,
    ],
)
def _combine_kernel(input_hbm, meta_hbm, ranks_hbm, hist_hbm, out_hbm,
                    hist_v, base_v, ev_v, rank_v, idx_v, in_v, out_v, sem):
    w = _wid()
    pltpu.sync_copy(hist_hbm, hist_v)
    pltpu.sync_copy(meta_hbm.at[pl.ds(w * CHUNK, CHUNK)], ev_v)
    pltpu.sync_copy(ranks_hbm.at[pl.ds(w * CHUNK, CHUNK)], rank_v)

    def hbody(wp, carry):
        pref, tot = carry
        hv = hist_v[pl.ds(wp * L, L)]
        pref = pref + jnp.where(wp < w, hv, 0)
        return pref, tot + hv

    zero = jnp.zeros((L,), jnp.int32)
    pref, tot = lax.fori_loop(0, NW, hbody, (zero, zero))
    offset = plsc.cumsum(tot) - tot
    base_v[...] = offset + pref

    def ibody(v, _):
        ev = ev_v[pl.ds(v * L, L)]
        bg = plsc.load_gather(base_v, [ev])
        idx_v[pl.ds(v * L, L)] = bg + rank_v[pl.ds(v * L, L)]
        return 0

    lax.fori_loop(0, CVECS, ibody, 0)

    def gbody(g, _):
        pltpu.async_copy(
            input_hbm.at[idx_v.at[pl.ds(g * 2 * GT, 2 * GT)]], in_v, sem
        ).wait()

        def sbody(s, _):
            i = s >> 7
            vv = (s & 127) * L
            a = in_v[2 * i, pl.ds(vv, L)]
            b = in_v[2 * i + 1, pl.ds(vv, L)]
            out_v[i, pl.ds(vv, L)] = a + b
            return 0

        lax.fori_loop(0, GT * (D // L), sbody, 0)
        pltpu.sync_copy(out_v, out_hbm.at[pl.ds(w * TOK_W + g * GT, GT)])
        return 0

    lax.fori_loop(0, NCHUNKS, gbody, 0)


def kernel(input_tensor, expert_metadata, expert_mapping, expert_locals):
    del expert_mapping, expert_locals
    meta = expert_metadata.reshape(-1).astype(jnp.int32)
    ranks, hist = _rank_kernel(meta)
    return _combine_kernel(input_tensor, meta, ranks, hist.reshape(-1))


# speedup vs baseline: 3.0161x; 1.3200x over previous
import functools

import jax
import jax.numpy as jnp
from jax import lax
from jax.experimental import pallas as pl
from jax.experimental.pallas import tpu as pltpu
from jax.experimental.pallas import tpu_sc as plsc

TOP_K = 2
NUM_EXPERTS = 16
T = 4096
D = 2048
N = T * TOP_K

NC, NS, L = 2, 16, 16
NW = NC * NS
CHUNK = N // NW
CVECS = CHUNK // L
TOK_W = T // NW
GT = 8
NCHUNKS = TOK_W // GT


def _wid():
    return lax.axis_index("s") * NC + lax.axis_index("c")


_mesh = plsc.VectorSubcoreMesh(core_axis_name="c", subcore_axis_name="s")


@functools.partial(
    pl.kernel,
    out_type=(
        jax.ShapeDtypeStruct((N,), jnp.int32),
        jax.ShapeDtypeStruct((NW, L), jnp.int32),
    ),
    mesh=_mesh,
    compiler_params=pltpu.CompilerParams(needs_layout_passes=False),
    scratch_types=[
        pltpu.VMEM((CHUNK,), jnp.int32),
        pltpu.VMEM((CHUNK,), jnp.int32),
        pltpu.VMEM((L,), jnp.int32),
    ],
)
def _rank_kernel(meta_hbm, ranks_hbm, hist_hbm, ev_v, rank_v, hist_v):
    w = _wid()
    pltpu.sync_copy(meta_hbm.at[pl.ds(w * CHUNK, CHUNK)], ev_v)
    hist_v[...] = jnp.zeros((L,), jnp.int32)

    def body(v, _):
        ev = ev_v[pl.ds(v * L, L)]
        carry = plsc.load_gather(hist_v, [ev])
        r = jnp.zeros((L,), jnp.int32)
        for e in range(NUM_EXPERTS):
            m = ev == e
            cs = plsc.cumsum(m.astype(jnp.int32))
            r = jnp.where(m, cs - 1, r)
        rank_v[pl.ds(v * L, L)] = carry + r
        plsc.addupdate_scatter(hist_v, [ev], jnp.ones((L,), jnp.int32))
        return 0

    lax.fori_loop(0, CVECS, body, 0)
    pltpu.sync_copy(rank_v, ranks_hbm.at[pl.ds(w * CHUNK, CHUNK)])
    pltpu.sync_copy(hist_v, hist_hbm.at[w])


@functools.partial(
    pl.kernel,
    out_type=jax.ShapeDtypeStruct((T, D), jnp.float32),
    mesh=_mesh,
    compiler_params=pltpu.CompilerParams(needs_layout_passes=False),
    scratch_types=[
        pltpu.VMEM((NW * L,), jnp.int32),
        pltpu.VMEM((L,), jnp.int32),
        pltpu.VMEM((CHUNK,), jnp.int32),
        pltpu.VMEM((CHUNK,), jnp.int32),
        pltpu.VMEM((CHUNK,), jnp.int32),
        pltpu.VMEM((2 * GT, D), jnp.float32),
        pltpu.VMEM((2 * GT, D), jnp.float32),
        pltpu.VMEM((GT, D), jnp.float32),
        pltpu.VMEM((GT, D), jnp.float32),
        pltpu.SemaphoreType.DMA,
        pltpu.SemaphoreType.DMA,
        pltpu.SemaphoreType.DMA,
        pltpu.SemaphoreType.DMA,
    ],
)
def _combine_kernel(input_hbm, meta_hbm, ranks_hbm, hist_hbm, out_hbm,
                    hist_v, base_v, ev_v, rank_v, idx_v,
                    in0_v, in1_v, out0_v, out1_v, gs0, gs1, os0, os1):
    w = _wid()
    pltpu.sync_copy(hist_hbm, hist_v)
    pltpu.sync_copy(meta_hbm.at[pl.ds(w * CHUNK, CHUNK)], ev_v)
    pltpu.sync_copy(ranks_hbm.at[pl.ds(w * CHUNK, CHUNK)], rank_v)

    def hbody(wp, carry):
        pref, tot = carry
        hv = hist_v[pl.ds(wp * L, L)]
        pref = pref + jnp.where(wp < w, hv, 0)
        return pref, tot + hv

    zero = jnp.zeros((L,), jnp.int32)
    pref, tot = lax.fori_loop(0, NW, hbody, (zero, zero))
    offset = plsc.cumsum(tot) - tot
    base_v[...] = offset + pref

    def ibody(v, _):
        ev = ev_v[pl.ds(v * L, L)]
        bg = plsc.load_gather(base_v, [ev])
        idx_v[pl.ds(v * L, L)] = bg + rank_v[pl.ds(v * L, L)]
        return 0

    lax.fori_loop(0, CVECS, ibody, 0)

    bufs = ((in0_v, out0_v, gs0, os0), (in1_v, out1_v, gs1, os1))

    def gather(g, inb, gs):
        return pltpu.async_copy(
            input_hbm.at[idx_v.at[pl.ds(g * 2 * GT, 2 * GT)]], inb, gs
        )

    gd = [gather(0, in0_v, gs0), gather(1, in1_v, gs1)]
    od = [None, None]
    for g in range(NCHUNKS):
        inb, outb, gs, os = bufs[g & 1]
        gd[g & 1].wait()
        if od[g & 1] is not None:
            od[g & 1].wait()

        def sbody(s, _, inb=inb, outb=outb):
            i = s >> 7
            vv = (s & 127) * L
            a = inb[2 * i, pl.ds(vv, L)]
            b = inb[2 * i + 1, pl.ds(vv, L)]
            outb[i, pl.ds(vv, L)] = a + b
            return 0

        lax.fori_loop(0, GT * (D // L), sbody, 0)
        od[g & 1] = pltpu.async_copy(
            outb, out_hbm.at[pl.ds(w * TOK_W + g * GT, GT)], os
        )
        if g + 2 < NCHUNKS:
            gd[g & 1] = gather(g + 2, inb, gs)
    od[0].wait()
    od[1].wait()


def kernel(input_tensor, expert_metadata, expert_mapping, expert_locals):
    del expert_mapping, expert_locals
    meta = expert_metadata.reshape(-1).astype(jnp.int32)
    ranks, hist = _rank_kernel(meta)
    return _combine_kernel(input_tensor, meta, ranks, hist.reshape(-1))


# speedup vs baseline: 4.7887x; 1.5877x over previous
import functools

import jax
import jax.numpy as jnp
from jax import lax
from jax.experimental import pallas as pl
from jax.experimental.pallas import tpu as pltpu
from jax.experimental.pallas import tpu_sc as plsc

TOP_K = 2
NUM_EXPERTS = 16
T = 4096
D = 2048
N = T * TOP_K

NC, NS, L = 2, 16, 16
NW = NC * NS
CHUNK = N // NW
CVECS = CHUNK // L
TOK_W = T // NW
GT = 8
NCHUNKS = TOK_W // GT


def _wid():
    return lax.axis_index("s") * NC + lax.axis_index("c")


_mesh = plsc.VectorSubcoreMesh(core_axis_name="c", subcore_axis_name="s")


@functools.partial(
    pl.kernel,
    out_type=(
        jax.ShapeDtypeStruct((N,), jnp.int32),
        jax.ShapeDtypeStruct((NW, L), jnp.int32),
    ),
    mesh=_mesh,
    compiler_params=pltpu.CompilerParams(needs_layout_passes=False),
    scratch_types=[
        pltpu.VMEM((CHUNK,), jnp.int32),
        pltpu.VMEM((CHUNK,), jnp.int32),
        pltpu.VMEM((L,), jnp.int32),
    ],
)
def _rank_kernel(meta_hbm, ranks_hbm, hist_hbm, ev_v, rank_v, hist_v):
    w = _wid()
    pltpu.sync_copy(meta_hbm.at[pl.ds(w * CHUNK, CHUNK)], ev_v)
    hist_v[...] = jnp.zeros((L,), jnp.int32)

    def body(v, _):
        ev = ev_v[pl.ds(v * L, L)]
        carry = plsc.load_gather(hist_v, [ev])
        r = jnp.zeros((L,), jnp.int32)
        for e in range(NUM_EXPERTS):
            m = ev == e
            cs = plsc.cumsum(m.astype(jnp.int32))
            r = jnp.where(m, cs - 1, r)
        rank_v[pl.ds(v * L, L)] = carry + r
        plsc.addupdate_scatter(hist_v, [ev], jnp.ones((L,), jnp.int32))
        return 0

    lax.fori_loop(0, CVECS, body, 0)
    pltpu.sync_copy(rank_v, ranks_hbm.at[pl.ds(w * CHUNK, CHUNK)])
    pltpu.sync_copy(hist_v, hist_hbm.at[w])


@functools.partial(
    pl.kernel,
    out_type=jax.ShapeDtypeStruct((T, D), jnp.float32),
    mesh=_mesh,
    compiler_params=pltpu.CompilerParams(needs_layout_passes=False),
    scratch_types=[
        pltpu.VMEM((NW * L,), jnp.int32),
        pltpu.VMEM((L,), jnp.int32),
        pltpu.VMEM((CHUNK,), jnp.int32),
        pltpu.VMEM((CHUNK,), jnp.int32),
        pltpu.VMEM((TOK_W,), jnp.int32),
        pltpu.VMEM((TOK_W,), jnp.int32),
        pltpu.VMEM((3, GT, D), jnp.float32),
        pltpu.VMEM((2, GT, D), jnp.float32),
        pltpu.SemaphoreType.DMA,
        pltpu.SemaphoreType.DMA,
        pltpu.SemaphoreType.DMA,
        pltpu.SemaphoreType.DMA,
        pltpu.SemaphoreType.DMA,
        pltpu.SemaphoreType.DMA,
        pltpu.SemaphoreType.DMA,
        pltpu.SemaphoreType.DMA,
    ],
)
def _combine_kernel(input_hbm, meta_hbm, ranks_hbm, hist_hbm, out_hbm,
                    hist_v, base_v, ev_v, rank_v, idx0_v, idx1_v,
                    outb_v, tmpb_v, ge0, ge1, ge2, go0, go1, os0, os1, os2):
    w = _wid()
    pltpu.sync_copy(hist_hbm, hist_v)
    pltpu.sync_copy(meta_hbm.at[pl.ds(w * CHUNK, CHUNK)], ev_v)
    pltpu.sync_copy(ranks_hbm.at[pl.ds(w * CHUNK, CHUNK)], rank_v)

    def hbody(wp, carry):
        pref, tot = carry
        hv = hist_v[pl.ds(wp * L, L)]
        pref = pref + jnp.where(wp < w, hv, 0)
        return pref, tot + hv

    zero = jnp.zeros((L,), jnp.int32)
    pref, tot = lax.fori_loop(0, NW, hbody, (zero, zero))
    offset = plsc.cumsum(tot) - tot
    base_v[...] = offset + pref

    lane = jnp.arange(L, dtype=jnp.int32)
    even = (lane & 1) == 0

    def ibody(v, _):
        ev = ev_v[pl.ds(v * L, L)]
        bg = plsc.load_gather(base_v, [ev])
        inv = bg + rank_v[pl.ds(v * L, L)]
        tok = v * (L // 2) + (lane >> 1)
        plsc.store_scatter(idx0_v, [tok], inv, mask=even)
        plsc.store_scatter(idx1_v, [tok], inv, mask=~even)
        return 0

    lax.fori_loop(0, CVECS, ibody, 0)

    ges, gos, oss = (ge0, ge1, ge2), (go0, go1), (os0, os1, os2)

    def gather_even(g):
        return pltpu.async_copy(
            input_hbm.at[idx0_v.at[pl.ds(g * GT, GT)]], outb_v.at[g % 3],
            ges[g % 3],
        )

    def gather_odd(g):
        return pltpu.async_copy(
            input_hbm.at[idx1_v.at[pl.ds(g * GT, GT)]], tmpb_v.at[g % 2],
            gos[g % 2],
        )

    ged = [None, None, None]
    god = [None, None]
    od = [None, None, None]
    ged[0] = gather_even(0)
    god[0] = gather_odd(0)
    ged[1] = gather_even(1)
    god[1] = gather_odd(1)
    UNROLL = 8
    for g in range(NCHUNKS):
        ged[g % 3].wait()
        god[g % 2].wait()
        outb = outb_v.at[g % 3]
        tmpb = tmpb_v.at[g % 2]
        for i in range(GT):
            def abody(s, _, i=i, outb=outb, tmpb=tmpb):
                for u in range(UNROLL):
                    c = (s * UNROLL + u) * L
                    plsc.addupdate(outb.at[i, pl.ds(c, L)], tmpb[i, pl.ds(c, L)])
                return 0

            lax.fori_loop(0, D // L // UNROLL, abody, 0)
        od[g % 3] = pltpu.async_copy(
            outb, out_hbm.at[pl.ds(w * TOK_W + g * GT, GT)], oss[g % 3]
        )
        if g + 2 < NCHUNKS:
            god[g % 2] = gather_odd(g + 2)
            if od[(g + 2) % 3] is not None:
                od[(g + 2) % 3].wait()
                od[(g + 2) % 3] = None
            ged[(g + 2) % 3] = gather_even(g + 2)
    for d in od:
        if d is not None:
            d.wait()


def kernel(input_tensor, expert_metadata, expert_mapping, expert_locals):
    del expert_mapping, expert_locals
    meta = expert_metadata.reshape(-1).astype(jnp.int32)
    ranks, hist = _rank_kernel(meta)
    return _combine_kernel(input_tensor, meta, ranks, hist.reshape(-1))


# speedup vs baseline: 4.9583x; 1.0354x over previous
import functools

import jax
import jax.numpy as jnp
from jax import lax
from jax.experimental import pallas as pl
from jax.experimental.pallas import tpu as pltpu
from jax.experimental.pallas import tpu_sc as plsc

TOP_K = 2
NUM_EXPERTS = 16
T = 4096
D = 2048
N = T * TOP_K

NC, NS, L = 2, 16, 16
NW = NC * NS
CHUNK = N // NW
CVECS = CHUNK // L
NVECS = N // L
TOK_W = T // NW
GT = 8
NCHUNKS = TOK_W // GT

_mesh = plsc.VectorSubcoreMesh(core_axis_name="c", subcore_axis_name="s")


@functools.partial(
    pl.kernel,
    out_type=jax.ShapeDtypeStruct((T, D), jnp.float32),
    mesh=_mesh,
    compiler_params=pltpu.CompilerParams(needs_layout_passes=False),
    scratch_types=[
        pltpu.VMEM((N,), jnp.int32),
        pltpu.VMEM((L,), jnp.int32),
        pltpu.VMEM((L,), jnp.int32),
        pltpu.VMEM((L,), jnp.int32),
        pltpu.VMEM((TOK_W,), jnp.int32),
        pltpu.VMEM((TOK_W,), jnp.int32),
        pltpu.VMEM((3, GT, D), jnp.float32),
        pltpu.VMEM((2, GT, D), jnp.float32),
        pltpu.SemaphoreType.DMA,
        pltpu.SemaphoreType.DMA,
        pltpu.SemaphoreType.DMA,
        pltpu.SemaphoreType.DMA,
        pltpu.SemaphoreType.DMA,
        pltpu.SemaphoreType.DMA,
        pltpu.SemaphoreType.DMA,
        pltpu.SemaphoreType.DMA,
    ],
)
def _combine_kernel(input_hbm, meta_hbm, out_hbm,
                    meta_v, cnt_v, cnt2_v, base_v, idx0_v, idx1_v,
                    outb_v, tmpb_v, ge0, ge1, ge2, go0, go1, os0, os1, os2):
    w = lax.axis_index("s") * NC + lax.axis_index("c")
    pltpu.sync_copy(meta_hbm, meta_v)
    zero = jnp.zeros((L,), jnp.int32)
    ones = jnp.ones((L,), jnp.int32)
    cnt_v[...] = zero
    cnt2_v[...] = zero

    def hbody(v, _):
        plsc.addupdate_scatter(cnt_v, [meta_v[pl.ds(v * L, L)]], ones)
        return 0

    lax.fori_loop(0, w * CVECS, hbody, 0)
    pref = cnt_v[...]
    lax.fori_loop(w * CVECS, NVECS, hbody, 0)
    tot = cnt_v[...]
    offset = plsc.cumsum(tot) - tot
    base_v[...] = offset + pref

    lane = jnp.arange(L, dtype=jnp.int32)
    even = (lane & 1) == 0

    def ibody(v, _):
        ev = meta_v[pl.ds((w * CVECS + v) * L, L)]
        carry = plsc.load_gather(cnt2_v, [ev])
        r = zero
        for e in range(NUM_EXPERTS):
            m = ev == e
            cs = plsc.cumsum(m.astype(jnp.int32))
            r = jnp.where(m, cs - 1, r)
        plsc.addupdate_scatter(cnt2_v, [ev], ones)
        inv = plsc.load_gather(base_v, [ev]) + carry + r
        tok = v * (L // 2) + (lane >> 1)
        plsc.store_scatter(idx0_v, [tok], inv, mask=even)
        plsc.store_scatter(idx1_v, [tok], inv, mask=~even)
        return 0

    lax.fori_loop(0, CVECS, ibody, 0)

    ges, gos, oss = (ge0, ge1, ge2), (go0, go1), (os0, os1, os2)

    def gather_even(g):
        return pltpu.async_copy(
            input_hbm.at[idx0_v.at[pl.ds(g * GT, GT)]], outb_v.at[g % 3],
            ges[g % 3],
        )

    def gather_odd(g):
        return pltpu.async_copy(
            input_hbm.at[idx1_v.at[pl.ds(g * GT, GT)]], tmpb_v.at[g % 2],
            gos[g % 2],
        )

    ged = [None, None, None]
    god = [None, None]
    od = [None, None, None]
    ged[0] = gather_even(0)
    god[0] = gather_odd(0)
    ged[1] = gather_even(1)
    god[1] = gather_odd(1)
    UNROLL = 8
    for g in range(NCHUNKS):
        ged[g % 3].wait()
        god[g % 2].wait()
        outb = outb_v.at[g % 3]
        tmpb = tmpb_v.at[g % 2]
        for i in range(GT):
            def abody(s, _, i=i, outb=outb, tmpb=tmpb):
                for u in range(UNROLL):
                    c = (s * UNROLL + u) * L
                    plsc.addupdate(outb.at[i, pl.ds(c, L)], tmpb[i, pl.ds(c, L)])
                return 0

            lax.fori_loop(0, D // L // UNROLL, abody, 0)
        od[g % 3] = pltpu.async_copy(
            outb, out_hbm.at[pl.ds(w * TOK_W + g * GT, GT)], oss[g % 3]
        )
        if g + 2 < NCHUNKS:
            god[g % 2] = gather_odd(g + 2)
            if od[(g + 2) % 3] is not None:
                od[(g + 2) % 3].wait()
                od[(g + 2) % 3] = None
            ged[(g + 2) % 3] = gather_even(g + 2)
    for d in od:
        if d is not None:
            d.wait()


def kernel(input_tensor, expert_metadata, expert_mapping, expert_locals):
    del expert_mapping, expert_locals
    meta = expert_metadata.reshape(-1).astype(jnp.int32)
    return _combine_kernel(input_tensor, meta)


# speedup vs baseline: 5.4763x; 1.1045x over previous
import functools

import jax
import jax.numpy as jnp
from jax import lax
from jax.experimental import pallas as pl
from jax.experimental.pallas import tpu as pltpu
from jax.experimental.pallas import tpu_sc as plsc

TOP_K = 2
NUM_EXPERTS = 16
T = 4096
D = 2048
N = T * TOP_K

NC, NS, L = 2, 16, 16
NW = NC * NS
CHUNK = N // NW
CVECS = CHUNK // L
NVECS = N // L
TOK_W = T // NW
GT = 8
NCHUNKS = TOK_W // GT

_mesh = plsc.VectorSubcoreMesh(core_axis_name="c", subcore_axis_name="s")


@functools.partial(
    pl.kernel,
    out_type=jax.ShapeDtypeStruct((T, D), jnp.float32),
    mesh=_mesh,
    compiler_params=pltpu.CompilerParams(needs_layout_passes=False),
    scratch_types=[
        pltpu.VMEM((N,), jnp.int32),
        pltpu.VMEM((L,), jnp.int32),
        pltpu.VMEM((L,), jnp.int32),
        pltpu.VMEM((L,), jnp.int32),
        pltpu.VMEM((TOK_W,), jnp.int32),
        pltpu.VMEM((TOK_W,), jnp.int32),
        pltpu.VMEM((3, GT, D), jnp.float32),
        pltpu.VMEM((2, GT, D), jnp.float32),
        pltpu.SemaphoreType.DMA,
        pltpu.SemaphoreType.DMA,
        pltpu.SemaphoreType.DMA,
        pltpu.SemaphoreType.DMA,
        pltpu.SemaphoreType.DMA,
        pltpu.SemaphoreType.DMA,
        pltpu.SemaphoreType.DMA,
        pltpu.SemaphoreType.DMA,
    ],
)
def _combine_kernel(input_hbm, meta_hbm, out_hbm,
                    meta_v, cnt_v, cnt2_v, base_v, idx0_v, idx1_v,
                    outb_v, tmpb_v, ge0, ge1, ge2, go0, go1, os0, os1, os2):
    w = lax.axis_index("s") * NC + lax.axis_index("c")
    pltpu.sync_copy(meta_hbm, meta_v)
    zero = jnp.zeros((L,), jnp.int32)
    ones = jnp.ones((L,), jnp.int32)
    cnt_v[...] = zero
    cnt2_v[...] = zero

    def hbody(v, _):
        plsc.addupdate_scatter(cnt_v, [meta_v[pl.ds(v * L, L)]], ones)
        return 0

    lax.fori_loop(0, w * CVECS, hbody, 0)
    pref = cnt_v[...]
    lax.fori_loop(w * CVECS, NVECS, hbody, 0)
    tot = cnt_v[...]
    offset = plsc.cumsum(tot) - tot
    base_v[...] = offset + pref

    lane = jnp.arange(L, dtype=jnp.int32)
    even = (lane & 1) == 0

    def ibody(v, _):
        ev = meta_v[pl.ds((w * CVECS + v) * L, L)]
        carry = plsc.load_gather(cnt2_v, [ev])
        r = zero
        for e in range(NUM_EXPERTS):
            m = ev == e
            cs = plsc.cumsum(m.astype(jnp.int32))
            r = jnp.where(m, cs - 1, r)
        plsc.addupdate_scatter(cnt2_v, [ev], ones)
        inv = plsc.load_gather(base_v, [ev]) + carry + r
        tok = v * (L // 2) + (lane >> 1)
        plsc.store_scatter(idx0_v, [tok], inv, mask=even)
        plsc.store_scatter(idx1_v, [tok], inv, mask=~even)
        return 0

    lax.fori_loop(0, CVECS, ibody, 0)

    ges, gos, oss = (ge0, ge1, ge2), (go0, go1), (os0, os1, os2)

    def gather_even(g):
        return pltpu.async_copy(
            input_hbm.at[idx0_v.at[pl.ds(g * GT, GT)]], outb_v.at[g % 3],
            ges[g % 3],
        )

    def gather_odd(g):
        return pltpu.async_copy(
            input_hbm.at[idx1_v.at[pl.ds(g * GT, GT)]], tmpb_v.at[g % 2],
            gos[g % 2],
        )

    ged = [None, None, None]
    god = [None, None]
    od = [None, None, None]
    ged[0] = gather_even(0)
    god[0] = gather_odd(0)
    ged[1] = gather_even(1)
    god[1] = gather_odd(1)
    UNROLL = 8
    for g in range(NCHUNKS):
        ged[g % 3].wait()
        god[g % 2].wait()
        outb = outb_v.at[g % 3]
        tmpb = tmpb_v.at[g % 2]

        @plsc.parallel_loop(0, GT * (D // L), 1, unroll=UNROLL)
        def _(s, outb=outb, tmpb=tmpb):
            i = s >> 7
            c = (s & 127) * L
            plsc.addupdate(outb.at[i, pl.ds(c, L)], tmpb[i, pl.ds(c, L)])
        od[g % 3] = pltpu.async_copy(
            outb, out_hbm.at[pl.ds(w * TOK_W + g * GT, GT)], oss[g % 3]
        )
        if g + 2 < NCHUNKS:
            god[g % 2] = gather_odd(g + 2)
            if od[(g + 2) % 3] is not None:
                od[(g + 2) % 3].wait()
                od[(g + 2) % 3] = None
            ged[(g + 2) % 3] = gather_even(g + 2)
    for d in od:
        if d is not None:
            d.wait()


def kernel(input_tensor, expert_metadata, expert_mapping, expert_locals):
    del expert_mapping, expert_locals
    meta = expert_metadata.reshape(-1).astype(jnp.int32)
    return _combine_kernel(input_tensor, meta)


# speedup vs baseline: 5.5626x; 1.0158x over previous
import functools

import jax
import jax.numpy as jnp
from jax import lax
from jax.experimental import pallas as pl
from jax.experimental.pallas import tpu as pltpu
from jax.experimental.pallas import tpu_sc as plsc

TOP_K = 2
NUM_EXPERTS = 16
T = 4096
D = 2048
N = T * TOP_K

NC, NS, L = 2, 16, 16
NW = NC * NS
CHUNK = N // NW
CVECS = CHUNK // L
NVECS = N // L
TOK_W = T // NW
GT = 8
NCHUNKS = TOK_W // GT

_mesh = plsc.VectorSubcoreMesh(core_axis_name="c", subcore_axis_name="s")


@functools.partial(
    pl.kernel,
    out_type=jax.ShapeDtypeStruct((T, D), jnp.float32),
    mesh=_mesh,
    compiler_params=pltpu.CompilerParams(needs_layout_passes=False),
    scratch_types=[
        pltpu.VMEM((N,), jnp.int32),
        pltpu.VMEM((L,), jnp.int32),
        pltpu.VMEM((L,), jnp.int32),
        pltpu.VMEM((L,), jnp.int32),
        pltpu.VMEM((TOK_W,), jnp.int32),
        pltpu.VMEM((TOK_W,), jnp.int32),
        pltpu.VMEM((3, GT, D), jnp.float32),
        pltpu.VMEM((2, GT, D), jnp.float32),
        pltpu.SemaphoreType.DMA,
        pltpu.SemaphoreType.DMA,
        pltpu.SemaphoreType.DMA,
        pltpu.SemaphoreType.DMA,
        pltpu.SemaphoreType.DMA,
        pltpu.SemaphoreType.DMA,
        pltpu.SemaphoreType.DMA,
        pltpu.SemaphoreType.DMA,
    ],
)
def _combine_kernel(input_hbm, meta_hbm, out_hbm,
                    meta_v, cnt_v, cnt2_v, base_v, idx0_v, idx1_v,
                    outb_v, tmpb_v, ge0, ge1, ge2, go0, go1, os0, os1, os2):
    w = lax.axis_index("s") * NC + lax.axis_index("c")
    pltpu.sync_copy(meta_hbm, meta_v)
    zero = jnp.zeros((L,), jnp.int32)
    ones = jnp.ones((L,), jnp.int32)
    cnt_v[...] = zero
    cnt2_v[...] = zero

    @plsc.parallel_loop(0, w * CVECS, 1, unroll=4)
    def _(v):
        plsc.addupdate_scatter(cnt_v, [meta_v[pl.ds(v * L, L)]], ones)

    pref = cnt_v[...]

    @plsc.parallel_loop(w * CVECS, NVECS, 1, unroll=4)
    def _(v):
        plsc.addupdate_scatter(cnt_v, [meta_v[pl.ds(v * L, L)]], ones)

    tot = cnt_v[...]
    offset = plsc.cumsum(tot) - tot
    base_v[...] = offset + pref

    lane = jnp.arange(L, dtype=jnp.int32)
    even = (lane & 1) == 0

    def ibody(v, _):
        ev = meta_v[pl.ds((w * CVECS + v) * L, L)]
        carry = plsc.load_gather(cnt2_v, [ev])
        r = zero
        for e in range(NUM_EXPERTS):
            m = ev == e
            cs = plsc.cumsum(m.astype(jnp.int32))
            r = jnp.where(m, cs - 1, r)
        plsc.addupdate_scatter(cnt2_v, [ev], ones)
        inv = plsc.load_gather(base_v, [ev]) + carry + r
        tok = v * (L // 2) + (lane >> 1)
        plsc.store_scatter(idx0_v, [tok], inv, mask=even)
        plsc.store_scatter(idx1_v, [tok], inv, mask=~even)
        return 0

    lax.fori_loop(0, CVECS, ibody, 0)

    ges, gos, oss = (ge0, ge1, ge2), (go0, go1), (os0, os1, os2)

    def gather_even(g):
        return pltpu.async_copy(
            input_hbm.at[idx0_v.at[pl.ds(g * GT, GT)]], outb_v.at[g % 3],
            ges[g % 3],
        )

    def gather_odd(g):
        return pltpu.async_copy(
            input_hbm.at[idx1_v.at[pl.ds(g * GT, GT)]], tmpb_v.at[g % 2],
            gos[g % 2],
        )

    ged = [None, None, None]
    god = [None, None]
    od = [None, None, None]
    ged[0] = gather_even(0)
    god[0] = gather_odd(0)
    ged[1] = gather_even(1)
    god[1] = gather_odd(1)
    UNROLL = 16
    for g in range(NCHUNKS):
        ged[g % 3].wait()
        god[g % 2].wait()
        outb = outb_v.at[g % 3]
        tmpb = tmpb_v.at[g % 2]

        @plsc.parallel_loop(0, GT * (D // L), 1, unroll=UNROLL)
        def _(s, outb=outb, tmpb=tmpb):
            i = s >> 7
            c = (s & 127) * L
            plsc.addupdate(outb.at[i, pl.ds(c, L)], tmpb[i, pl.ds(c, L)])
        od[g % 3] = pltpu.async_copy(
            outb, out_hbm.at[pl.ds(w * TOK_W + g * GT, GT)], oss[g % 3]
        )
        if g + 2 < NCHUNKS:
            god[g % 2] = gather_odd(g + 2)
            if od[(g + 2) % 3] is not None:
                od[(g + 2) % 3].wait()
                od[(g + 2) % 3] = None
            ged[(g + 2) % 3] = gather_even(g + 2)
    for d in od:
        if d is not None:
            d.wait()


def kernel(input_tensor, expert_metadata, expert_mapping, expert_locals):
    del expert_mapping, expert_locals
    meta = expert_metadata.reshape(-1).astype(jnp.int32)
    return _combine_kernel(input_tensor, meta)


# speedup vs baseline: 5.6468x; 1.0151x over previous
import functools

import jax
import jax.numpy as jnp
from jax import lax
from jax.experimental import pallas as pl
from jax.experimental.pallas import tpu as pltpu
from jax.experimental.pallas import tpu_sc as plsc

TOP_K = 2
NUM_EXPERTS = 16
T = 4096
D = 2048
N = T * TOP_K

NC, NS, L = 2, 16, 16
NW = NC * NS
CHUNK = N // NW
CVECS = CHUNK // L
NVECS = N // L
TOK_W = T // NW
GT = 8
NCHUNKS = TOK_W // GT

_mesh = plsc.VectorSubcoreMesh(core_axis_name="c", subcore_axis_name="s")


@functools.partial(
    pl.kernel,
    out_type=jax.ShapeDtypeStruct((T, D), jnp.float32),
    mesh=_mesh,
    compiler_params=pltpu.CompilerParams(needs_layout_passes=False),
    scratch_types=[
        pltpu.VMEM((N,), jnp.int32),
        pltpu.VMEM((L,), jnp.int32),
        pltpu.VMEM((L,), jnp.int32),
        pltpu.VMEM((L,), jnp.int32),
        pltpu.VMEM((TOK_W,), jnp.int32),
        pltpu.VMEM((TOK_W,), jnp.int32),
        pltpu.VMEM((4, GT, D), jnp.float32),
        pltpu.VMEM((3, GT, D), jnp.float32),
        pltpu.SemaphoreType.DMA,
        pltpu.SemaphoreType.DMA,
        pltpu.SemaphoreType.DMA,
        pltpu.SemaphoreType.DMA,
        pltpu.SemaphoreType.DMA,
        pltpu.SemaphoreType.DMA,
        pltpu.SemaphoreType.DMA,
        pltpu.SemaphoreType.DMA,
        pltpu.SemaphoreType.DMA,
        pltpu.SemaphoreType.DMA,
        pltpu.SemaphoreType.DMA,
    ],
)
def _combine_kernel(input_hbm, meta_hbm, out_hbm,
                    meta_v, cnt_v, cnt2_v, base_v, idx0_v, idx1_v,
                    outb_v, tmpb_v, ge0, ge1, ge2, ge3, go0, go1, go2,
                    os0, os1, os2, os3):
    w = lax.axis_index("s") * NC + lax.axis_index("c")
    pltpu.sync_copy(meta_hbm, meta_v)
    zero = jnp.zeros((L,), jnp.int32)
    ones = jnp.ones((L,), jnp.int32)
    cnt_v[...] = zero
    cnt2_v[...] = zero

    @plsc.parallel_loop(0, w * CVECS, 1, unroll=4)
    def _(v):
        plsc.addupdate_scatter(cnt_v, [meta_v[pl.ds(v * L, L)]], ones)

    pref = cnt_v[...]

    @plsc.parallel_loop(w * CVECS, NVECS, 1, unroll=4)
    def _(v):
        plsc.addupdate_scatter(cnt_v, [meta_v[pl.ds(v * L, L)]], ones)

    tot = cnt_v[...]
    offset = plsc.cumsum(tot) - tot
    base_v[...] = offset + pref

    lane = jnp.arange(L, dtype=jnp.int32)
    even = (lane & 1) == 0

    def ibody(v, _):
        ev = meta_v[pl.ds((w * CVECS + v) * L, L)]
        carry = plsc.load_gather(cnt2_v, [ev])
        r = zero
        for e in range(NUM_EXPERTS):
            m = ev == e
            cs = plsc.cumsum(m.astype(jnp.int32))
            r = jnp.where(m, cs - 1, r)
        plsc.addupdate_scatter(cnt2_v, [ev], ones)
        inv = plsc.load_gather(base_v, [ev]) + carry + r
        tok = v * (L // 2) + (lane >> 1)
        plsc.store_scatter(idx0_v, [tok], inv, mask=even)
        plsc.store_scatter(idx1_v, [tok], inv, mask=~even)
        return 0

    lax.fori_loop(0, CVECS, ibody, 0)

    ges, gos, oss = (ge0, ge1, ge2, ge3), (go0, go1, go2), (os0, os1, os2, os3)
    NEB, NOB = 4, 3
    AHEAD = 3

    def gather_even(g):
        return pltpu.async_copy(
            input_hbm.at[idx0_v.at[pl.ds(g * GT, GT)]], outb_v.at[g % NEB],
            ges[g % NEB],
        )

    def gather_odd(g):
        return pltpu.async_copy(
            input_hbm.at[idx1_v.at[pl.ds(g * GT, GT)]], tmpb_v.at[g % NOB],
            gos[g % NOB],
        )

    ged = [None] * NEB
    god = [None] * NOB
    od = [None] * NEB
    for g in range(AHEAD):
        ged[g % NEB] = gather_even(g)
        god[g % NOB] = gather_odd(g)
    UNROLL = 16
    for g in range(NCHUNKS):
        ged[g % NEB].wait()
        god[g % NOB].wait()
        outb = outb_v.at[g % NEB]
        tmpb = tmpb_v.at[g % NOB]

        @plsc.parallel_loop(0, GT * (D // L), 1, unroll=UNROLL)
        def _(s, outb=outb, tmpb=tmpb):
            i = s >> 7
            c = (s & 127) * L
            plsc.addupdate(outb.at[i, pl.ds(c, L)], tmpb[i, pl.ds(c, L)])
        od[g % NEB] = pltpu.async_copy(
            outb, out_hbm.at[pl.ds(w * TOK_W + g * GT, GT)], oss[g % NEB]
        )
        if g + AHEAD < NCHUNKS:
            god[g % NOB] = gather_odd(g + AHEAD)
            if od[(g + AHEAD) % NEB] is not None:
                od[(g + AHEAD) % NEB].wait()
                od[(g + AHEAD) % NEB] = None
            ged[(g + AHEAD) % NEB] = gather_even(g + AHEAD)
    for d in od:
        if d is not None:
            d.wait()


def kernel(input_tensor, expert_metadata, expert_mapping, expert_locals):
    del expert_mapping, expert_locals
    meta = expert_metadata.reshape(-1).astype(jnp.int32)
    return _combine_kernel(input_tensor, meta)
